# Initial kernel scaffold; baseline (speedup 1.0000x reference)
import jax
import jax.numpy as jnp
from jax.experimental import pallas as pl


def kernel(x, target):
    raise NotImplementedError("write your pallas kernel here")


# speedup vs baseline: 1.5186x; 1.5186x over previous
import functools

import jax
import jax.numpy as jnp
from jax.experimental import pallas as pl

_SMOOTHING = 0.1


def _bce_body(x_ref, tgt_ref, o_ref, *, nsteps, inv_n, on_value, off_value):
    i = pl.program_id(0)

    @pl.when(i == 0)
    def _init():
        o_ref[...] = jnp.zeros_like(o_ref)

    xb = x_ref[...]
    tgt = tgt_ref[...]
    col = jax.lax.broadcasted_iota(jnp.int32, xb.shape, 1)
    t = jnp.where(col == tgt, on_value, off_value)
    sp = jnp.maximum(xb, 0.0) + jnp.log1p(jnp.exp(-jnp.abs(xb)))
    o_ref[...] = o_ref[...] + jnp.sum(sp - xb * t)

    @pl.when(i == nsteps - 1)
    def _finish():
        o_ref[...] = o_ref[...] * inv_n


def kernel(x, target):
    b, c = x.shape
    off_value = _SMOOTHING / c
    on_value = 1.0 - _SMOOTHING + off_value
    tgt = target.reshape(b, 1).astype(jnp.int32)

    block_rows = 512
    nsteps = b // block_rows

    out = pl.pallas_call(
        functools.partial(
            _bce_body,
            nsteps=nsteps,
            inv_n=1.0 / (b * c),
            on_value=float(on_value),
            off_value=float(off_value),
        ),
        grid=(nsteps,),
        in_specs=[
            pl.BlockSpec((block_rows, c), lambda i: (i, 0)),
            pl.BlockSpec((block_rows, 1), lambda i: (i, 0)),
        ],
        out_specs=pl.BlockSpec((1, 1), lambda i: (0, 0)),
        out_shape=jax.ShapeDtypeStruct((1, 1), jnp.float32),
    )(x, tgt)
    return out[0, 0]
